# baseline (speedup 1.0000x reference)
import jax
import jax.numpy as jnp
from jax.experimental import pallas as pl
from jax.experimental.pallas import tpu as pltpu

PRE_SEQ_LEN = 128
HIDDEN = 1024
OUT_DIM = 24 * 2 * 1024
TOKENS = 8 * 128
BLK = 1024
NBLK = OUT_DIM // BLK


def _body(prefix_ref, emb_ref, w1_ref, b1_ref, w2_ref, b2_ref, out_ref,
          h_ref, oh_ref, t_ref):
    p = pl.program_id(0)
    j = pl.program_id(1)

    @pl.when((p == 0) & (j == 0))
    def _init():
        h_ref[...] = jnp.tanh(
            jnp.dot(emb_ref[...], w1_ref[...],
                    preferred_element_type=jnp.float32) + b1_ref[...]
        ).astype(jnp.bfloat16)
        row_ids = jax.lax.broadcasted_iota(jnp.int32, (TOKENS, PRE_SEQ_LEN), 1)
        oh_ref[...] = (prefix_ref[...] == row_ids).astype(jnp.bfloat16)

    @pl.when(p == 0)
    def _compute_table():
        t = jnp.dot(h_ref[...], w2_ref[...].astype(jnp.bfloat16),
                    preferred_element_type=jnp.float32)
        t_ref[:, pl.ds(j * BLK, BLK)] = t.astype(jnp.bfloat16)

    @pl.when(p == 1)
    def _expand():
        out_ref[...] = jnp.dot(oh_ref[...], t_ref[:, pl.ds(j * BLK, BLK)],
                               preferred_element_type=jnp.float32) + b2_ref[...]


def kernel(prefix, emb, W1, b1, W2, b2):
    prefix2d = prefix.reshape(TOKENS, 1).astype(jnp.int32)
    b1r = b1.reshape(1, HIDDEN)
    b2r = b2.reshape(1, OUT_DIM)
    out = pl.pallas_call(
        _body,
        grid=(2, NBLK),
        in_specs=[
            pl.BlockSpec((TOKENS, 1), lambda p, j: (0, 0)),
            pl.BlockSpec((PRE_SEQ_LEN, HIDDEN), lambda p, j: (0, 0)),
            pl.BlockSpec((HIDDEN, HIDDEN), lambda p, j: (0, 0)),
            pl.BlockSpec((1, HIDDEN), lambda p, j: (0, 0)),
            pl.BlockSpec((HIDDEN, BLK),
                         lambda p, j: (0, jnp.where(p == 0, j, NBLK - 1))),
            pl.BlockSpec((1, BLK), lambda p, j: (0, j)),
        ],
        out_specs=pl.BlockSpec((TOKENS, BLK),
                               lambda p, j: (0, jnp.where(p == 0, 0, j))),
        out_shape=jax.ShapeDtypeStruct((TOKENS, OUT_DIM), jnp.float32),
        scratch_shapes=[
            pltpu.VMEM((PRE_SEQ_LEN, HIDDEN), jnp.bfloat16),
            pltpu.VMEM((TOKENS, PRE_SEQ_LEN), jnp.bfloat16),
            pltpu.VMEM((PRE_SEQ_LEN, OUT_DIM), jnp.bfloat16),
        ],
        compiler_params=pltpu.CompilerParams(
            dimension_semantics=("arbitrary", "arbitrary"),
        ),
    )(prefix2d, emb, W1, b1r, W2, b2r)
    return out.reshape(prefix.shape[0], prefix.shape[1], OUT_DIM)


# speedup vs baseline: 1.1153x; 1.1153x over previous
import jax
import jax.numpy as jnp
from jax.experimental import pallas as pl
from jax.experimental.pallas import tpu as pltpu

PRE_SEQ_LEN = 128
HIDDEN = 1024
OUT_DIM = 24 * 2 * 1024
TOKENS = 8 * 128
BLK = 2048
NBLK = OUT_DIM // BLK


def _body(prefix_ref, emb_ref, w1_ref, b1_ref, w2_ref, b2_ref, out_ref,
          h_ref, oh_ref, t_ref):
    p = pl.program_id(0)
    j = pl.program_id(1)

    @pl.when((p == 0) & (j == 0))
    def _init():
        h_ref[...] = jnp.tanh(
            jnp.dot(emb_ref[...], w1_ref[...],
                    preferred_element_type=jnp.float32) + b1_ref[...]
        ).astype(jnp.bfloat16)
        row_ids = jax.lax.broadcasted_iota(jnp.int32, (TOKENS, PRE_SEQ_LEN), 1)
        oh_ref[...] = (prefix_ref[...] == row_ids).astype(jnp.bfloat16)

    @pl.when(p == 0)
    def _compute_table():
        t = jnp.dot(h_ref[...], w2_ref[...].astype(jnp.bfloat16),
                    preferred_element_type=jnp.float32)
        t_ref[:, pl.ds(j * BLK, BLK)] = t.astype(jnp.bfloat16)

    @pl.when(p == 1)
    def _expand():
        out_ref[...] = jnp.dot(oh_ref[...], t_ref[:, pl.ds(j * BLK, BLK)],
                               preferred_element_type=jnp.float32) + b2_ref[...]


def kernel(prefix, emb, W1, b1, W2, b2):
    prefix2d = prefix.reshape(TOKENS, 1).astype(jnp.int32)
    b1r = b1.reshape(1, HIDDEN)
    b2r = b2.reshape(1, OUT_DIM)
    out = pl.pallas_call(
        _body,
        grid=(2, NBLK),
        in_specs=[
            pl.BlockSpec((TOKENS, 1), lambda p, j: (0, 0)),
            pl.BlockSpec((PRE_SEQ_LEN, HIDDEN), lambda p, j: (0, 0)),
            pl.BlockSpec((HIDDEN, HIDDEN), lambda p, j: (0, 0)),
            pl.BlockSpec((1, HIDDEN), lambda p, j: (0, 0)),
            pl.BlockSpec((HIDDEN, BLK),
                         lambda p, j: (0, jnp.where(p == 0, j, NBLK - 1))),
            pl.BlockSpec((1, BLK), lambda p, j: (0, j)),
        ],
        out_specs=pl.BlockSpec((TOKENS, BLK),
                               lambda p, j: (0, jnp.where(p == 0, 0, j))),
        out_shape=jax.ShapeDtypeStruct((TOKENS, OUT_DIM), jnp.float32),
        scratch_shapes=[
            pltpu.VMEM((PRE_SEQ_LEN, HIDDEN), jnp.bfloat16),
            pltpu.VMEM((TOKENS, PRE_SEQ_LEN), jnp.bfloat16),
            pltpu.VMEM((PRE_SEQ_LEN, OUT_DIM), jnp.bfloat16),
        ],
        compiler_params=pltpu.CompilerParams(
            dimension_semantics=("arbitrary", "arbitrary"),
        ),
    )(prefix2d, emb, W1, b1r, W2, b2r)
    return out.reshape(prefix.shape[0], prefix.shape[1], OUT_DIM)
